# baseline (speedup 1.0000x reference)
import jax
import jax.numpy as jnp
from jax import lax
from jax.experimental import pallas as pl
from jax.experimental.pallas import tpu as pltpu

_EPS = 0.001
_CLIP_LO = 1e-07
_CLIP_HI = 10000000.0

_ROWS_PER_BLOCK = 9216
_CHUNK_ROWS = 144


def _make_loss_kernel(n_steps, rows_per_block, chunk_rows):
    n_chunks = rows_per_block // chunk_rows

    def _loss_block_kernel(p_ref, t_ref, s0_ref, s1_ref, n0_ref,
                           acc0_ref, acc1_ref, accn_ref):
        @pl.when(pl.program_id(0) == 0)
        def _init():
            acc0_ref[...] = jnp.zeros_like(acc0_ref)
            acc1_ref[...] = jnp.zeros_like(acc1_ref)
            accn_ref[...] = jnp.zeros_like(accn_ref)

        w = p_ref.shape[-1]
        sub = chunk_rows // 8

        def body(i, carry):
            a0, a1, an = carry
            off = i * chunk_rows
            p = p_ref[pl.ds(off, chunk_rows), :]
            t = t_ref[pl.ds(off, chunk_rows), :]
            e = p - t
            s = jnp.minimum(jnp.maximum(e * e, _CLIP_LO), _CLIP_HI)
            m0f = jnp.where(t > _EPS, 1.0, 0.0)
            s0c = s * m0f
            a0 = a0 + jnp.sum(s0c.reshape(sub, 8, w), axis=0)
            a1 = a1 + jnp.sum(s.reshape(sub, 8, w), axis=0)
            an = an + jnp.sum(m0f.reshape(sub, 8, w), axis=0)
            return a0, a1, an

        zeros = jnp.zeros((8, w), jnp.float32)
        a0, a1, an = lax.fori_loop(0, n_chunks, body, (zeros, zeros, zeros))
        acc0_ref[...] += a0
        acc1_ref[...] += a1
        accn_ref[...] += an

        @pl.when(pl.program_id(0) == n_steps - 1)
        def _finish():
            s0_ref[...] = jnp.sum(acc0_ref[...]).reshape(1, 1, 1)
            s1_ref[...] = jnp.sum(acc1_ref[...] - acc0_ref[...]).reshape(1, 1, 1)
            n0_ref[...] = jnp.sum(accn_ref[...].astype(jnp.int32)).reshape(1, 1, 1)

    return _loss_block_kernel


def kernel(pred, truth):
    n_total = pred.size
    p2 = pred.reshape(-1, pred.shape[-1])
    t2 = truth.reshape(-1, truth.shape[-1])
    rows, cols = p2.shape
    n_steps = rows // _ROWS_PER_BLOCK

    in_spec = pl.BlockSpec((_ROWS_PER_BLOCK, cols), lambda i: (i, 0))
    out_spec = pl.BlockSpec((1, 1, 1), lambda i: (0, 0, 0))

    s0, s1, n0 = pl.pallas_call(
        _make_loss_kernel(n_steps, _ROWS_PER_BLOCK, _CHUNK_ROWS),
        grid=(n_steps,),
        in_specs=[in_spec, in_spec],
        out_specs=[out_spec, out_spec, out_spec],
        out_shape=[
            jax.ShapeDtypeStruct((1, 1, 1), jnp.float32),
            jax.ShapeDtypeStruct((1, 1, 1), jnp.float32),
            jax.ShapeDtypeStruct((1, 1, 1), jnp.int32),
        ],
        scratch_shapes=[
            pltpu.VMEM((8, cols), jnp.float32),
            pltpu.VMEM((8, cols), jnp.float32),
            pltpu.VMEM((8, cols), jnp.float32),
        ],
        compiler_params=pltpu.CompilerParams(
            dimension_semantics=("arbitrary",),
        ),
    )(p2, t2)

    s0 = s0[0, 0, 0]
    s1 = s1[0, 0, 0]
    n0 = n0[0, 0, 0].astype(jnp.float32)
    n1 = jnp.float32(n_total) - n0
    mean1 = s1 / jnp.maximum(n1, 1.0)
    mean0 = jnp.where(n0 > 0, s0 / jnp.maximum(n0, 1.0), mean1)
    return mean0 + mean1


# speedup vs baseline: 1.0229x; 1.0229x over previous
import jax
import jax.numpy as jnp
from jax import lax
from jax.experimental import pallas as pl
from jax.experimental.pallas import tpu as pltpu

_EPS = 0.001
_CLIP_LO = 1e-07
_CLIP_HI = 10000000.0

_ROWS_PER_BLOCK = 6144
_CHUNK_ROWS = 96


def _make_loss_kernel(n_steps, rows_per_block, chunk_rows):
    n_chunks = rows_per_block // chunk_rows

    def _loss_block_kernel(p_ref, t_ref, s0_ref, s1_ref, n0_ref,
                           acc0_ref, acc1_ref, accn_ref):
        @pl.when(pl.program_id(0) == 0)
        def _init():
            acc0_ref[...] = jnp.zeros_like(acc0_ref)
            acc1_ref[...] = jnp.zeros_like(acc1_ref)
            accn_ref[...] = jnp.zeros_like(accn_ref)

        w = p_ref.shape[-1]
        sub = chunk_rows // 8

        def body(i, carry):
            a0, a1, an = carry
            off = i * chunk_rows
            p = p_ref[pl.ds(off, chunk_rows), :]
            t = t_ref[pl.ds(off, chunk_rows), :]
            e = p - t
            s = jnp.minimum(jnp.maximum(e * e, _CLIP_LO), _CLIP_HI)
            m0f = jnp.where(t > _EPS, 1.0, 0.0)
            s0c = s * m0f
            a0 = a0 + jnp.sum(s0c.reshape(sub, 8, w), axis=0)
            a1 = a1 + jnp.sum(s.reshape(sub, 8, w), axis=0)
            an = an + jnp.sum(m0f.reshape(sub, 8, w), axis=0)
            return a0, a1, an

        zeros = jnp.zeros((8, w), jnp.float32)
        a0, a1, an = lax.fori_loop(0, n_chunks, body, (zeros, zeros, zeros))
        acc0_ref[...] += a0
        acc1_ref[...] += a1
        accn_ref[...] += an

        @pl.when(pl.program_id(0) == n_steps - 1)
        def _finish():
            s0_ref[...] = jnp.sum(acc0_ref[...]).reshape(1, 1, 1)
            s1_ref[...] = jnp.sum(acc1_ref[...] - acc0_ref[...]).reshape(1, 1, 1)
            n0_ref[...] = jnp.sum(accn_ref[...].astype(jnp.int32)).reshape(1, 1, 1)

    return _loss_block_kernel


def kernel(pred, truth):
    n_total = pred.size
    p2 = pred.reshape(-1, pred.shape[-1])
    t2 = truth.reshape(-1, truth.shape[-1])
    rows, cols = p2.shape
    n_steps = rows // _ROWS_PER_BLOCK

    in_spec = pl.BlockSpec((_ROWS_PER_BLOCK, cols), lambda i: (i, 0))
    out_spec = pl.BlockSpec((1, 1, 1), lambda i: (0, 0, 0))

    s0, s1, n0 = pl.pallas_call(
        _make_loss_kernel(n_steps, _ROWS_PER_BLOCK, _CHUNK_ROWS),
        grid=(n_steps,),
        in_specs=[in_spec, in_spec],
        out_specs=[out_spec, out_spec, out_spec],
        out_shape=[
            jax.ShapeDtypeStruct((1, 1, 1), jnp.float32),
            jax.ShapeDtypeStruct((1, 1, 1), jnp.float32),
            jax.ShapeDtypeStruct((1, 1, 1), jnp.int32),
        ],
        scratch_shapes=[
            pltpu.VMEM((8, cols), jnp.float32),
            pltpu.VMEM((8, cols), jnp.float32),
            pltpu.VMEM((8, cols), jnp.float32),
        ],
        compiler_params=pltpu.CompilerParams(
            dimension_semantics=("arbitrary",),
        ),
    )(p2, t2)

    s0 = s0[0, 0, 0]
    s1 = s1[0, 0, 0]
    n0 = n0[0, 0, 0].astype(jnp.float32)
    n1 = jnp.float32(n_total) - n0
    mean1 = s1 / jnp.maximum(n1, 1.0)
    mean0 = jnp.where(n0 > 0, s0 / jnp.maximum(n0, 1.0), mean1)
    return mean0 + mean1
